# baseline (speedup 1.0000x reference)
import functools

import jax
import jax.numpy as jnp
from jax import lax
from jax.experimental import pallas as pl
from jax.experimental.pallas import tpu as pltpu
from jax.experimental.pallas import tpu_sc as plsc

EMBED = 1024
FEATS = 32768
BATCH = 8192
K = 32
CW = 256
NCHUNK = FEATS // CW
ROWS = 512
FT = 512
NFT = FEATS // FT
NCAND = 192


def _encode_kernel(embed_ref, w_ref, scores_ref, tau_ref, chunks_ref,
                   cmax_ref):
    j = pl.program_id(1)

    x = embed_ref[...]
    w = w_ref[...]
    s = jax.lax.dot_general(
        x, w, (((1,), (1,)), ((), ())),
        preferred_element_type=jnp.float32)
    scores_ref[...] = s
    neg0 = jnp.float32(-jnp.inf)
    c0 = jnp.max(s[:, :CW], axis=1, keepdims=True)
    c1 = jnp.max(s[:, CW:], axis=1, keepdims=True)
    li = jax.lax.broadcasted_iota(jnp.int32, (ROWS, NCHUNK), 1)
    contrib = jnp.where(li == 2 * j, c0,
                        jnp.where(li == 2 * j + 1, c1, neg0))

    @pl.when(j == 0)
    def _init():
        cmax_ref[...] = contrib

    @pl.when(j > 0)
    def _acc():
        cmax_ref[...] = jnp.maximum(cmax_ref[...], contrib)

    @pl.when(j == NFT - 1)
    def _tau():
        ch_iota = jax.lax.broadcasted_iota(jnp.int32, (ROWS, NCHUNK), 1)
        k_iota = jax.lax.broadcasted_iota(jnp.int32, (ROWS, K), 1)
        neg = jnp.float32(-jnp.inf)

        def body(k, carry):
            cmax, tau, cacc = carry
            m = jnp.max(cmax, axis=1)
            c = jnp.argmax(cmax, axis=1).astype(jnp.int32)
            cmax = jnp.where(ch_iota == c[:, None], neg, cmax)
            cacc = jnp.where(k_iota == k, c[:, None], cacc)
            return cmax, m, cacc

        carry0 = (cmax_ref[...], jnp.zeros((ROWS,), jnp.float32),
                  jnp.zeros((ROWS, K), jnp.int32))
        _, tau, cacc = jax.lax.fori_loop(0, K, body, carry0, unroll=False)
        tau_ref[...] = tau[:, None]
        chunks_ref[...] = cacc


def _encode(embed0_bf, enc_w_bf):
    grid = (BATCH // ROWS, NFT)
    return pl.pallas_call(
        _encode_kernel,
        grid=grid,
        in_specs=[
            pl.BlockSpec((ROWS, EMBED), lambda i, j: (i, 0)),
            pl.BlockSpec((FT, EMBED), lambda i, j: (j, 0)),
        ],
        out_specs=[
            pl.BlockSpec((ROWS, FT), lambda i, j: (i, j)),
            pl.BlockSpec((ROWS, 1), lambda i, j: (i, 0)),
            pl.BlockSpec((ROWS, K), lambda i, j: (i, 0)),
        ],
        out_shape=[
            jax.ShapeDtypeStruct((BATCH, FEATS), jnp.float32),
            jax.ShapeDtypeStruct((BATCH, 1), jnp.float32),
            jax.ShapeDtypeStruct((BATCH, K), jnp.int32),
        ],
        scratch_shapes=[
            pltpu.VMEM((ROWS, NCHUNK), jnp.float32),
        ],
        compiler_params=pltpu.CompilerParams(
            dimension_semantics=("arbitrary", "arbitrary"),
        ),
    )(embed0_bf, enc_w_bf)


def _select_kernel(cv_ref, cf_ref, w_ref, f_ref):
    R = cv_ref.shape[0]
    cv = cv_ref[...]
    cf = cf_ref[...]
    pos_iota = jax.lax.broadcasted_iota(jnp.int32, (R, NCAND), 1)
    k_iota = jax.lax.broadcasted_iota(jnp.int32, (R, K), 1)
    neg = jnp.float32(-jnp.inf)

    cf0 = cf[:, :128]
    cf1 = cf[:, 128:]

    def body(k, carry):
        cv, wacc, facc = carry
        m = jnp.max(cv, axis=1)
        p = jnp.argmax(cv, axis=1).astype(jnp.int32)
        p0 = jnp.minimum(p, 127)
        p1 = jnp.clip(p - 128, 0, NCAND - 129)
        f0 = jnp.take_along_axis(cf0, p0[:, None], axis=1)[:, 0]
        f1 = jnp.take_along_axis(cf1, p1[:, None], axis=1)[:, 0]
        fid = jnp.where(p < 128, f0, f1)
        cv = jnp.where(pos_iota == p[:, None], neg, cv)
        ksel = k_iota == k
        wacc = jnp.where(ksel, m[:, None], wacc)
        facc = jnp.where(ksel, fid[:, None], facc)
        return cv, wacc, facc

    carry0 = (cv, jnp.zeros((R, K), jnp.float32), jnp.zeros((R, K), jnp.int32))
    _, wacc, facc = jax.lax.fori_loop(0, K, body, carry0, unroll=False)
    w_ref[...] = wacc
    f_ref[...] = facc


def _select(cand_v, cand_f):
    R = 1024
    return pl.pallas_call(
        _select_kernel,
        grid=(BATCH // R,),
        in_specs=[
            pl.BlockSpec((R, NCAND), lambda i: (i, 0)),
            pl.BlockSpec((R, NCAND), lambda i: (i, 0)),
        ],
        out_specs=[
            pl.BlockSpec((R, K), lambda i: (i, 0)),
            pl.BlockSpec((R, K), lambda i: (i, 0)),
        ],
        out_shape=[
            jax.ShapeDtypeStruct((BATCH, K), jnp.float32),
            jax.ShapeDtypeStruct((BATCH, K), jnp.int32),
        ],
    )(cand_v, cand_f)


def _make_compact():
    info = plsc.get_sparse_core_info()
    NW = info.num_cores * info.num_subcores
    RPW = BATCH // NW
    NG = CW // 16
    mesh = plsc.VectorSubcoreMesh(core_axis_name="c", subcore_axis_name="s")

    @functools.partial(
        pl.kernel, mesh=mesh,
        out_type=[
            jax.ShapeDtypeStruct((BATCH * NCAND,), jnp.float32),
            jax.ShapeDtypeStruct((BATCH * NCAND,), jnp.int32),
        ],
        scratch_types=[
            pltpu.VMEM((RPW * K + 16,), jnp.int32),
            pltpu.VMEM((K,), jnp.int32),
            pltpu.VMEM((K, CW), jnp.float32),
            pltpu.VMEM((K * CW + 16,), jnp.float32),
            pltpu.VMEM((RPW + 16,), jnp.float32),
            pltpu.VMEM((K * NG + 16,), jnp.float32),
            pltpu.VMEM((NCAND,), jnp.float32),
            pltpu.VMEM((NCAND,), jnp.int32),
            pltpu.SMEM((1,), jnp.int32),
            pltpu.SemaphoreType.DMA,
        ],
    )
    def compact(scores_hbm, chunks_hbm, tau_hbm, cv_hbm, cf_hbm,
                cid_v, idx_v, data_v, flat_v, tau_v, sg_v, cand_v, candf_v,
                off_s, sem):
        wid = lax.axis_index("s") * 2 + lax.axis_index("c")
        r0 = wid * RPW
        pltpu.sync_copy(tau_hbm.at[pl.ds(r0, RPW)], tau_v.at[pl.ds(0, RPW)])
        pltpu.sync_copy(chunks_hbm.at[pl.ds(r0 * K, RPW * K)],
                        cid_v.at[pl.ds(0, RPW * K)])
        neginf = jnp.full((16,), -jnp.inf, jnp.float32)
        zero16 = jnp.zeros((16,), jnp.int32)
        pow2 = jax.lax.shift_left(jnp.int32(1), lax.iota(jnp.int32, 16))

        def row_body(r, _):
            row = r0 + r
            for t in range(K // 16):
                cid = cid_v[pl.ds(r * K + t * 16, 16)]
                idx_v[pl.ds(t * 16, 16)] = row * NCHUNK + cid
            pltpu.async_copy(scores_hbm.at[idx_v], data_v, sem).wait()
            tau_s = tau_v[pl.ds(r, 16)][0]
            for s in range(K):
                m = data_v[s, pl.ds(0, 16)]
                flat_v[pl.ds(s * CW, 16)] = m
                for t in range(1, NG):
                    v = data_v[s, pl.ds(t * 16, 16)]
                    flat_v[pl.ds(s * CW + t * 16, 16)] = v
                    m = jnp.maximum(m, v)
                sg_v[pl.ds(s * 16, 16)] = m
            for t in range(NCAND // 16):
                cand_v[pl.ds(t * 16, 16)] = neginf
            off_s[0] = 0

            def scan_cell(j):
                s = j // 16
                l = j - s * 16
                fbase = cid_v[pl.ds(r * K + s, 16)][0] * CW + l
                for i in range(NG):
                    vs = flat_v[pl.ds(s * CW + i * 16 + l, 16)][0]

                    @pl.when(vs >= tau_s)
                    def _append(vs=vs, i=i, fbase=fbase):
                        off = off_s[0]
                        cand_v[pl.ds(off, 16)] = (
                            jnp.zeros((16,), jnp.float32) + vs)
                        candf_v[pl.ds(off, 16)] = (
                            zero16 + fbase + i * 16)
                        off_s[0] = jnp.minimum(off + 1, NCAND - 17)

            tau_vec = jnp.zeros((16,), jnp.float32) + tau_s

            def cell_body(s, _):
                sgv = sg_v[pl.ds(s * 16, 16)]
                bits = jnp.where(sgv >= tau_vec, pow2, zero16)
                b = bits[0]
                for t in range(1, 16):
                    b = b + bits[t]

                @pl.when(b > 0)
                def _walk(s=s, b=b):
                    for l in range(16):
                        @pl.when((b >> l) % 2 != 0)
                        def _scan(s=s, l=l):
                            scan_cell(s * 16 + l)

                return ()

            lax.fori_loop(0, K, cell_body, (), unroll=False)
            cand_v[pl.ds(off_s[0], 16)] = neginf
            pltpu.sync_copy(cand_v, cv_hbm.at[pl.ds(row * NCAND, NCAND)])
            pltpu.sync_copy(candf_v, cf_hbm.at[pl.ds(row * NCAND, NCAND)])
            return ()

        lax.fori_loop(0, RPW, row_body, (), unroll=False)

    return compact


def _make_decode():
    info = plsc.get_sparse_core_info()
    NW = info.num_cores * info.num_subcores
    RPW = BATCH // NW
    mesh = plsc.VectorSubcoreMesh(core_axis_name="c", subcore_axis_name="s")

    @functools.partial(
        pl.kernel, mesh=mesh,
        out_type=jax.ShapeDtypeStruct((BATCH * EMBED,), jnp.float32),
        scratch_types=[
            pltpu.VMEM((K,), jnp.int32),
            pltpu.VMEM((RPW * K + 16,), jnp.int32),
            pltpu.VMEM((RPW * K + 16,), jnp.float32),
            pltpu.VMEM((K, EMBED), jnp.float32),
            pltpu.VMEM((EMBED,), jnp.float32),
            pltpu.VMEM((EMBED,), jnp.float32),
            pltpu.SemaphoreType.DMA,
        ],
    )
    def decode(dec_hbm, feats_hbm, w_hbm, bias_hbm, out_hbm,
               fid_v, fall_v, wall_v, rows_v, bias_v, acc_v, sem):
        wid = lax.axis_index("s") * 2 + lax.axis_index("c")
        r0 = wid * RPW
        pltpu.sync_copy(bias_hbm, bias_v)
        pltpu.sync_copy(feats_hbm.at[pl.ds(r0 * K, RPW * K)],
                        fall_v.at[pl.ds(0, RPW * K)])
        pltpu.sync_copy(w_hbm.at[pl.ds(r0 * K, RPW * K)],
                        wall_v.at[pl.ds(0, RPW * K)])

        def row_body(r, _):
            row = r0 + r
            for t in range(K // 16):
                fid_v[pl.ds(t * 16, 16)] = fall_v[pl.ds(r * K + t * 16, 16)]
            pltpu.async_copy(dec_hbm.at[fid_v], rows_v, sem).wait()
            NB = 16
            for jb in range(EMBED // (16 * NB)):
                base = jb * 16 * NB
                accs = tuple(bias_v[pl.ds(base + t * 16, 16)]
                             for t in range(NB))

                def k_body(k, accs, base=base):
                    wk = (jnp.zeros((16,), jnp.float32)
                          + wall_v[pl.ds(r * K + k, 16)][0])
                    return tuple(
                        accs[t] + wk * rows_v[k, pl.ds(base + t * 16, 16)]
                        for t in range(NB))

                accs = lax.fori_loop(0, K, k_body, accs, unroll=False)
                for t in range(NB):
                    acc_v[pl.ds(base + t * 16, 16)] = accs[t]

            ss = jnp.zeros((16,), jnp.float32)
            for j in range(EMBED // 16):
                v = acc_v[pl.ds(j * 16, 16)]
                ss = ss + v * v
            tot_s = ss[0]
            for t in range(1, 16):
                tot_s = tot_s + ss[t]
            tot = jnp.zeros((16,), jnp.float32) + tot_s
            tot = jnp.maximum(tot, jnp.float32(1e-24))
            i = lax.bitcast_convert_type(tot, jnp.int32)
            y = lax.bitcast_convert_type(
                jnp.int32(0x5F3759DF) - (i >> 1), jnp.float32)
            half = jnp.float32(0.5) * tot
            for _ in range(4):
                y = y * (jnp.float32(1.5) - half * y * y)
            for j in range(EMBED // 16):
                acc_v[pl.ds(j * 16, 16)] = acc_v[pl.ds(j * 16, 16)] * y
            pltpu.sync_copy(acc_v, out_hbm.at[pl.ds(row * EMBED, EMBED)])
            return ()

        lax.fori_loop(0, RPW, row_body, (), unroll=False)

    return decode


def kernel(embed, enc_bias, enc_weight, dec_lookup):
    embed0 = (embed - enc_bias).astype(jnp.bfloat16)
    w_bf = enc_weight.astype(jnp.bfloat16)
    scores, tau, topchunks = _encode(embed0, w_bf)
    scoresC = scores.reshape(BATCH * NCHUNK, CW)
    cand_v, cand_f = _make_compact()(
        scoresC, topchunks.reshape(BATCH * K), tau.reshape(BATCH))
    weights, feats = _select(cand_v.reshape(BATCH, NCAND),
                             cand_f.reshape(BATCH, NCAND))
    embed1 = _make_decode()(dec_lookup, feats.reshape(BATCH * K),
                            weights.reshape(BATCH * K), enc_bias)
    return embed1.reshape(BATCH, EMBED)


# speedup vs baseline: 3.0862x; 3.0862x over previous
import functools

import jax
import jax.numpy as jnp
from jax import lax
from jax.experimental import pallas as pl
from jax.experimental.pallas import tpu as pltpu
from jax.experimental.pallas import tpu_sc as plsc

EMBED = 1024
FEATS = 32768
BATCH = 8192
K = 32
CW = 256
NCHUNK = FEATS // CW
ROWS = 512
FT = 512
NFT = FEATS // FT
NCAND = 128


def _encode_kernel(embed_ref, w_ref, scores_ref, tau_ref, chunks_ref,
                   cmax_ref):
    j = pl.program_id(1)

    x = embed_ref[...]
    w = w_ref[...]
    s = jax.lax.dot_general(
        x, w, (((1,), (1,)), ((), ())),
        preferred_element_type=jnp.float32)
    scores_ref[...] = s
    neg0 = jnp.float32(-jnp.inf)
    c0 = jnp.max(s[:, :CW], axis=1, keepdims=True)
    c1 = jnp.max(s[:, CW:], axis=1, keepdims=True)
    li = jax.lax.broadcasted_iota(jnp.int32, (ROWS, NCHUNK), 1)
    contrib = jnp.where(li == 2 * j, c0,
                        jnp.where(li == 2 * j + 1, c1, neg0))

    @pl.when(j == 0)
    def _init():
        cmax_ref[...] = contrib

    @pl.when(j > 0)
    def _acc():
        cmax_ref[...] = jnp.maximum(cmax_ref[...], contrib)

    @pl.when(j == NFT - 1)
    def _tau():
        ch_iota = jax.lax.broadcasted_iota(jnp.int32, (ROWS, NCHUNK), 1)
        k_iota = jax.lax.broadcasted_iota(jnp.int32, (ROWS, K), 1)
        neg = jnp.float32(-jnp.inf)

        def body(k, carry):
            cmax, tau, cacc = carry
            m = jnp.max(cmax, axis=1)
            c = jnp.argmax(cmax, axis=1).astype(jnp.int32)
            cmax = jnp.where(ch_iota == c[:, None], neg, cmax)
            cacc = jnp.where(k_iota == k, c[:, None], cacc)
            return cmax, m, cacc

        carry0 = (cmax_ref[...], jnp.zeros((ROWS,), jnp.float32),
                  jnp.zeros((ROWS, K), jnp.int32))
        _, tau, cacc = jax.lax.fori_loop(0, K, body, carry0, unroll=False)
        tau_ref[...] = tau[:, None]
        chunks_ref[...] = cacc


def _encode(embed0_bf, enc_w_bf):
    grid = (BATCH // ROWS, NFT)
    return pl.pallas_call(
        _encode_kernel,
        grid=grid,
        in_specs=[
            pl.BlockSpec((ROWS, EMBED), lambda i, j: (i, 0)),
            pl.BlockSpec((FT, EMBED), lambda i, j: (j, 0)),
        ],
        out_specs=[
            pl.BlockSpec((ROWS, FT), lambda i, j: (i, j)),
            pl.BlockSpec((ROWS, 1), lambda i, j: (i, 0)),
            pl.BlockSpec((ROWS, K), lambda i, j: (i, 0)),
        ],
        out_shape=[
            jax.ShapeDtypeStruct((BATCH, FEATS), jnp.float32),
            jax.ShapeDtypeStruct((BATCH, 1), jnp.float32),
            jax.ShapeDtypeStruct((BATCH, K), jnp.int32),
        ],
        scratch_shapes=[
            pltpu.VMEM((ROWS, NCHUNK), jnp.float32),
        ],
        compiler_params=pltpu.CompilerParams(
            dimension_semantics=("arbitrary", "arbitrary"),
        ),
    )(embed0_bf, enc_w_bf)


def _select_kernel(cv_ref, cf_ref, w_ref, f_ref):
    R = cv_ref.shape[0]
    cv = cv_ref[...]
    cf = cf_ref[...]
    pos_iota = jax.lax.broadcasted_iota(jnp.int32, (R, NCAND), 1)
    k_iota = jax.lax.broadcasted_iota(jnp.int32, (R, K), 1)
    neg = jnp.float32(-jnp.inf)

    def body(k, carry):
        cv, wacc, facc = carry
        m = jnp.max(cv, axis=1)
        p = jnp.argmax(cv, axis=1).astype(jnp.int32)
        fid = jnp.take_along_axis(cf, p[:, None], axis=1)[:, 0]
        cv = jnp.where(pos_iota == p[:, None], neg, cv)
        ksel = k_iota == k
        wacc = jnp.where(ksel, m[:, None], wacc)
        facc = jnp.where(ksel, fid[:, None], facc)
        return cv, wacc, facc

    carry0 = (cv, jnp.zeros((R, K), jnp.float32), jnp.zeros((R, K), jnp.int32))
    _, wacc, facc = jax.lax.fori_loop(0, K, body, carry0, unroll=False)
    w_ref[...] = wacc
    f_ref[...] = facc


def _select(cand_v, cand_f):
    R = 1024
    return pl.pallas_call(
        _select_kernel,
        grid=(BATCH // R,),
        in_specs=[
            pl.BlockSpec((R, NCAND), lambda i: (i, 0)),
            pl.BlockSpec((R, NCAND), lambda i: (i, 0)),
        ],
        out_specs=[
            pl.BlockSpec((R, K), lambda i: (i, 0)),
            pl.BlockSpec((R, K), lambda i: (i, 0)),
        ],
        out_shape=[
            jax.ShapeDtypeStruct((BATCH, K), jnp.float32),
            jax.ShapeDtypeStruct((BATCH, K), jnp.int32),
        ],
    )(cand_v, cand_f)


def _make_compact():
    info = plsc.get_sparse_core_info()
    NW = info.num_cores * info.num_subcores
    RPW = BATCH // NW
    NG = CW // 16
    mesh = plsc.VectorSubcoreMesh(core_axis_name="c", subcore_axis_name="s")

    @functools.partial(
        pl.kernel, mesh=mesh,
        out_type=[
            jax.ShapeDtypeStruct((BATCH * NCAND,), jnp.float32),
            jax.ShapeDtypeStruct((BATCH * NCAND,), jnp.int32),
        ],
        scratch_types=[
            pltpu.VMEM((RPW * K + 16,), jnp.int32),
            pltpu.VMEM((K,), jnp.int32),
            pltpu.VMEM((K, CW), jnp.float32),
            pltpu.VMEM((K * CW + 16,), jnp.float32),
            pltpu.VMEM((RPW + 16,), jnp.float32),
            pltpu.VMEM((K * NG + 16,), jnp.float32),
            pltpu.VMEM((NCAND,), jnp.float32),
            pltpu.VMEM((NCAND,), jnp.int32),
            pltpu.SMEM((1,), jnp.int32),
            pltpu.SemaphoreType.DMA,
        ],
    )
    def compact(scores_hbm, chunks_hbm, tau_hbm, cv_hbm, cf_hbm,
                cid_v, idx_v, data_v, flat_v, tau_v, sg_v, cand_v, candf_v,
                off_s, sem):
        wid = lax.axis_index("s") * 2 + lax.axis_index("c")
        r0 = wid * RPW
        pltpu.sync_copy(tau_hbm.at[pl.ds(r0, RPW)], tau_v.at[pl.ds(0, RPW)])
        pltpu.sync_copy(chunks_hbm.at[pl.ds(r0 * K, RPW * K)],
                        cid_v.at[pl.ds(0, RPW * K)])
        neginf = jnp.full((16,), -jnp.inf, jnp.float32)
        zero16 = jnp.zeros((16,), jnp.int32)
        pow2 = jax.lax.shift_left(jnp.int32(1), lax.iota(jnp.int32, 16))

        def row_body(r, _):
            row = r0 + r
            for t in range(K // 16):
                cid = cid_v[pl.ds(r * K + t * 16, 16)]
                idx_v[pl.ds(t * 16, 16)] = row * NCHUNK + cid
            pltpu.async_copy(scores_hbm.at[idx_v], data_v, sem).wait()
            tau_s = tau_v[pl.ds(r, 16)][0]
            for s in range(K):
                m = data_v[s, pl.ds(0, 16)]
                flat_v[pl.ds(s * CW, 16)] = m
                for t in range(1, NG):
                    v = data_v[s, pl.ds(t * 16, 16)]
                    flat_v[pl.ds(s * CW + t * 16, 16)] = v
                    m = jnp.maximum(m, v)
                sg_v[pl.ds(s * 16, 16)] = m
            for t in range(NCAND // 16):
                cand_v[pl.ds(t * 16, 16)] = neginf
            off_s[0] = 0

            def scan_cell(j):
                s = j // 16
                l = j - s * 16
                fbase = cid_v[pl.ds(r * K + s, 16)][0] * CW + l
                for i in range(NG):
                    vs = flat_v[pl.ds(s * CW + i * 16 + l, 16)][0]

                    @pl.when(vs >= tau_s)
                    def _append(vs=vs, i=i, fbase=fbase):
                        off = off_s[0]
                        cand_v[pl.ds(off, 16)] = (
                            jnp.zeros((16,), jnp.float32) + vs)
                        candf_v[pl.ds(off, 16)] = (
                            zero16 + fbase + i * 16)
                        off_s[0] = jnp.minimum(off + 1, NCAND - 17)

            def cell_body(j4, _):
                for u in range(4):
                    j = j4 * 4 + u
                    sgs = sg_v[pl.ds(j, 16)][0]

                    @pl.when(sgs >= tau_s)
                    def _scan(j=j):
                        scan_cell(j)

                return ()

            lax.fori_loop(0, K * 4, cell_body, (), unroll=False)
            cand_v[pl.ds(off_s[0], 16)] = neginf
            pltpu.sync_copy(cand_v, cv_hbm.at[pl.ds(row * NCAND, NCAND)])
            pltpu.sync_copy(candf_v, cf_hbm.at[pl.ds(row * NCAND, NCAND)])
            return ()

        lax.fori_loop(0, RPW, row_body, (), unroll=False)

    return compact


def _make_decode():
    info = plsc.get_sparse_core_info()
    NW = info.num_cores * info.num_subcores
    RPW = BATCH // NW
    mesh = plsc.VectorSubcoreMesh(core_axis_name="c", subcore_axis_name="s")

    @functools.partial(
        pl.kernel, mesh=mesh,
        out_type=jax.ShapeDtypeStruct((BATCH * EMBED,), jnp.float32),
        scratch_types=[
            pltpu.VMEM((K,), jnp.int32),
            pltpu.VMEM((RPW * K + 16,), jnp.int32),
            pltpu.VMEM((RPW * K + 16,), jnp.float32),
            pltpu.VMEM((K, EMBED), jnp.float32),
            pltpu.VMEM((EMBED,), jnp.float32),
            pltpu.VMEM((EMBED,), jnp.float32),
            pltpu.SemaphoreType.DMA,
        ],
    )
    def decode(dec_hbm, feats_hbm, w_hbm, bias_hbm, out_hbm,
               fid_v, fall_v, wall_v, rows_v, bias_v, acc_v, sem):
        wid = lax.axis_index("s") * 2 + lax.axis_index("c")
        r0 = wid * RPW
        pltpu.sync_copy(bias_hbm, bias_v)
        pltpu.sync_copy(feats_hbm.at[pl.ds(r0 * K, RPW * K)],
                        fall_v.at[pl.ds(0, RPW * K)])
        pltpu.sync_copy(w_hbm.at[pl.ds(r0 * K, RPW * K)],
                        wall_v.at[pl.ds(0, RPW * K)])

        def row_body(r, _):
            row = r0 + r
            for t in range(K // 16):
                fid_v[pl.ds(t * 16, 16)] = fall_v[pl.ds(r * K + t * 16, 16)]
            pltpu.async_copy(dec_hbm.at[fid_v], rows_v, sem).wait()
            NB = 16
            for jb in range(EMBED // (16 * NB)):
                base = jb * 16 * NB
                accs = tuple(bias_v[pl.ds(base + t * 16, 16)]
                             for t in range(NB))

                def k_body(k, accs, base=base):
                    wk = (jnp.zeros((16,), jnp.float32)
                          + wall_v[pl.ds(r * K + k, 16)][0])
                    return tuple(
                        accs[t] + wk * rows_v[k, pl.ds(base + t * 16, 16)]
                        for t in range(NB))

                accs = lax.fori_loop(0, K, k_body, accs, unroll=False)
                for t in range(NB):
                    acc_v[pl.ds(base + t * 16, 16)] = accs[t]

            ss = jnp.zeros((16,), jnp.float32)
            for j in range(EMBED // 16):
                v = acc_v[pl.ds(j * 16, 16)]
                ss = ss + v * v
            tot_s = ss[0]
            for t in range(1, 16):
                tot_s = tot_s + ss[t]
            tot = jnp.zeros((16,), jnp.float32) + tot_s
            tot = jnp.maximum(tot, jnp.float32(1e-24))
            i = lax.bitcast_convert_type(tot, jnp.int32)
            y = lax.bitcast_convert_type(
                jnp.int32(0x5F3759DF) - (i >> 1), jnp.float32)
            half = jnp.float32(0.5) * tot
            for _ in range(4):
                y = y * (jnp.float32(1.5) - half * y * y)
            for j in range(EMBED // 16):
                acc_v[pl.ds(j * 16, 16)] = acc_v[pl.ds(j * 16, 16)] * y
            pltpu.sync_copy(acc_v, out_hbm.at[pl.ds(row * EMBED, EMBED)])
            return ()

        lax.fori_loop(0, RPW, row_body, (), unroll=False)

    return decode


def kernel(embed, enc_bias, enc_weight, dec_lookup):
    embed0 = (embed - enc_bias).astype(jnp.bfloat16)
    w_bf = enc_weight.astype(jnp.bfloat16)
    scores, tau, topchunks = _encode(embed0, w_bf)
    scoresC = scores.reshape(BATCH * NCHUNK, CW)
    cand_v, cand_f = _make_compact()(
        scoresC, topchunks.reshape(BATCH * K), tau.reshape(BATCH))
    weights, feats = _select(cand_v.reshape(BATCH, NCAND),
                             cand_f.reshape(BATCH, NCAND))
    embed1 = _make_decode()(dec_lookup, feats.reshape(BATCH * K),
                            weights.reshape(BATCH * K), enc_bias)
    return embed1.reshape(BATCH, EMBED)


# speedup vs baseline: 3.4121x; 1.1056x over previous
import functools

import jax
import jax.numpy as jnp
from jax import lax
from jax.experimental import pallas as pl
from jax.experimental.pallas import tpu as pltpu
from jax.experimental.pallas import tpu_sc as plsc

EMBED = 1024
FEATS = 32768
BATCH = 8192
K = 32
CW = 256
NCHUNK = FEATS // CW
ROWS = 512
FT = 512
NFT = FEATS // FT
NCAND = 128


def _encode_kernel(embed_ref, w_ref, scores_ref, tau_ref, chunks_ref,
                   cmax_ref):
    j = pl.program_id(1)

    x = embed_ref[...]
    w = w_ref[...]
    s = jax.lax.dot_general(
        x, w, (((1,), (1,)), ((), ())),
        preferred_element_type=jnp.float32)
    scores_ref[...] = s
    neg0 = jnp.float32(-jnp.inf)
    c0 = jnp.max(s[:, :CW], axis=1, keepdims=True)
    c1 = jnp.max(s[:, CW:], axis=1, keepdims=True)
    li = jax.lax.broadcasted_iota(jnp.int32, (ROWS, NCHUNK), 1)
    contrib = jnp.where(li == 2 * j, c0,
                        jnp.where(li == 2 * j + 1, c1, neg0))

    @pl.when(j == 0)
    def _init():
        cmax_ref[...] = contrib

    @pl.when(j > 0)
    def _acc():
        cmax_ref[...] = jnp.maximum(cmax_ref[...], contrib)

    @pl.when(j == NFT - 1)
    def _tau():
        ch_iota = jax.lax.broadcasted_iota(jnp.int32, (ROWS, NCHUNK), 1)
        k_iota = jax.lax.broadcasted_iota(jnp.int32, (ROWS, K), 1)
        neg = jnp.float32(-jnp.inf)

        def body(k, carry):
            cmax, tau, cacc = carry
            m = jnp.max(cmax, axis=1)
            c = jnp.argmax(cmax, axis=1).astype(jnp.int32)
            cmax = jnp.where(ch_iota == c[:, None], neg, cmax)
            cacc = jnp.where(k_iota == k, c[:, None], cacc)
            return cmax, m, cacc

        carry0 = (cmax_ref[...], jnp.zeros((ROWS,), jnp.float32),
                  jnp.zeros((ROWS, K), jnp.int32))
        _, tau, cacc = jax.lax.fori_loop(0, K, body, carry0, unroll=False)
        tau_ref[...] = tau[:, None]
        chunks_ref[...] = cacc


def _encode(embed0_bf, enc_w_bf):
    grid = (BATCH // ROWS, NFT)
    return pl.pallas_call(
        _encode_kernel,
        grid=grid,
        in_specs=[
            pl.BlockSpec((ROWS, EMBED), lambda i, j: (i, 0)),
            pl.BlockSpec((FT, EMBED), lambda i, j: (j, 0)),
        ],
        out_specs=[
            pl.BlockSpec((ROWS, FT), lambda i, j: (i, j)),
            pl.BlockSpec((ROWS, 1), lambda i, j: (i, 0)),
            pl.BlockSpec((ROWS, K), lambda i, j: (i, 0)),
        ],
        out_shape=[
            jax.ShapeDtypeStruct((BATCH, FEATS), jnp.float32),
            jax.ShapeDtypeStruct((BATCH, 1), jnp.float32),
            jax.ShapeDtypeStruct((BATCH, K), jnp.int32),
        ],
        scratch_shapes=[
            pltpu.VMEM((ROWS, NCHUNK), jnp.float32),
        ],
        compiler_params=pltpu.CompilerParams(
            dimension_semantics=("arbitrary", "arbitrary"),
        ),
    )(embed0_bf, enc_w_bf)


def _select_kernel(cv_ref, cf_ref, w_ref, f_ref):
    R = cv_ref.shape[0]
    cv = cv_ref[...]
    cf = cf_ref[...]
    pos_iota = jax.lax.broadcasted_iota(jnp.int32, (R, NCAND), 1)
    k_iota = jax.lax.broadcasted_iota(jnp.int32, (R, K), 1)
    neg = jnp.float32(-jnp.inf)

    def body(k, carry):
        cv, wacc, facc = carry
        m = jnp.max(cv, axis=1)
        p = jnp.argmax(cv, axis=1).astype(jnp.int32)
        fid = jnp.take_along_axis(cf, p[:, None], axis=1)[:, 0]
        cv = jnp.where(pos_iota == p[:, None], neg, cv)
        ksel = k_iota == k
        wacc = jnp.where(ksel, m[:, None], wacc)
        facc = jnp.where(ksel, fid[:, None], facc)
        return cv, wacc, facc

    carry0 = (cv, jnp.zeros((R, K), jnp.float32), jnp.zeros((R, K), jnp.int32))
    _, wacc, facc = jax.lax.fori_loop(0, K, body, carry0, unroll=False)
    w_ref[...] = wacc
    f_ref[...] = facc


def _select(cand_v, cand_f):
    R = 1024
    return pl.pallas_call(
        _select_kernel,
        grid=(BATCH // R,),
        in_specs=[
            pl.BlockSpec((R, NCAND), lambda i: (i, 0)),
            pl.BlockSpec((R, NCAND), lambda i: (i, 0)),
        ],
        out_specs=[
            pl.BlockSpec((R, K), lambda i: (i, 0)),
            pl.BlockSpec((R, K), lambda i: (i, 0)),
        ],
        out_shape=[
            jax.ShapeDtypeStruct((BATCH, K), jnp.float32),
            jax.ShapeDtypeStruct((BATCH, K), jnp.int32),
        ],
    )(cand_v, cand_f)


def _make_compact():
    info = plsc.get_sparse_core_info()
    NW = info.num_cores * info.num_subcores
    RPW = BATCH // NW
    NG = CW // 16
    mesh = plsc.VectorSubcoreMesh(core_axis_name="c", subcore_axis_name="s")

    @functools.partial(
        pl.kernel, mesh=mesh,
        out_type=[
            jax.ShapeDtypeStruct((BATCH * NCAND,), jnp.float32),
            jax.ShapeDtypeStruct((BATCH * NCAND,), jnp.int32),
        ],
        scratch_types=[
            pltpu.VMEM((RPW * K + 16,), jnp.int32),
            pltpu.VMEM((K,), jnp.int32),
            pltpu.VMEM((K, CW), jnp.float32),
            pltpu.VMEM((K * CW + 16,), jnp.float32),
            pltpu.VMEM((RPW + 16,), jnp.float32),
            pltpu.VMEM((K * NG + 16,), jnp.float32),
            pltpu.VMEM((NCAND,), jnp.float32),
            pltpu.VMEM((NCAND,), jnp.int32),
            pltpu.SMEM((1,), jnp.int32),
            pltpu.SemaphoreType.DMA,
        ],
    )
    def compact(scores_hbm, chunks_hbm, tau_hbm, cv_hbm, cf_hbm,
                cid_v, idx_v, data_v, flat_v, tau_v, sg_v, cand_v, candf_v,
                off_s, sem):
        wid = lax.axis_index("s") * 2 + lax.axis_index("c")
        r0 = wid * RPW
        pltpu.sync_copy(tau_hbm.at[pl.ds(r0, RPW)], tau_v.at[pl.ds(0, RPW)])
        pltpu.sync_copy(chunks_hbm.at[pl.ds(r0 * K, RPW * K)],
                        cid_v.at[pl.ds(0, RPW * K)])
        neginf = jnp.full((16,), -jnp.inf, jnp.float32)
        zero16 = jnp.zeros((16,), jnp.int32)
        pow2 = jax.lax.shift_left(jnp.int32(1), lax.iota(jnp.int32, 16))

        def row_body(r, _):
            row = r0 + r
            for t in range(K // 16):
                cid = cid_v[pl.ds(r * K + t * 16, 16)]
                idx_v[pl.ds(t * 16, 16)] = row * NCHUNK + cid
            pltpu.async_copy(scores_hbm.at[idx_v], data_v, sem).wait()
            tau_s = tau_v[pl.ds(r, 16)][0]
            for s in range(K):
                m = data_v[s, pl.ds(0, 16)]
                flat_v[pl.ds(s * CW, 16)] = m
                for t in range(1, NG):
                    v = data_v[s, pl.ds(t * 16, 16)]
                    flat_v[pl.ds(s * CW + t * 16, 16)] = v
                    m = jnp.maximum(m, v)
                sg_v[pl.ds(s * 16, 16)] = m
            for t in range(NCAND // 16):
                cand_v[pl.ds(t * 16, 16)] = neginf
            off_s[0] = 0

            def scan_cell(j):
                s = j // 16
                l = j - s * 16
                fbase = cid_v[pl.ds(r * K + s, 16)][0] * CW + l
                for i in range(NG):
                    vs = flat_v[pl.ds(s * CW + i * 16 + l, 16)][0]

                    @pl.when(vs >= tau_s)
                    def _append(vs=vs, i=i, fbase=fbase):
                        off = off_s[0]
                        cand_v[pl.ds(off, 16)] = (
                            jnp.zeros((16,), jnp.float32) + vs)
                        candf_v[pl.ds(off, 16)] = (
                            zero16 + fbase + i * 16)
                        off_s[0] = jnp.minimum(off + 1, NCAND - 17)

            tau_vec = jnp.zeros((16,), jnp.float32) + tau_s

            def cell_body(s, _):
                sgv = sg_v[pl.ds(s * 16, 16)]
                bits = jnp.where(sgv >= tau_vec, pow2, zero16)
                b = bits[0]
                for t in range(1, 16):
                    b = b + bits[t]

                @pl.when(b > 0)
                def _walk(s=s, b=b):
                    def lane_body(l, _):
                        @pl.when((b >> l) % 2 != 0)
                        def _scan():
                            scan_cell(s * 16 + l)
                        return ()
                    lax.fori_loop(0, 16, lane_body, (), unroll=False)

                return ()

            lax.fori_loop(0, K, cell_body, (), unroll=False)
            cand_v[pl.ds(off_s[0], 16)] = neginf
            pltpu.sync_copy(cand_v, cv_hbm.at[pl.ds(row * NCAND, NCAND)])
            pltpu.sync_copy(candf_v, cf_hbm.at[pl.ds(row * NCAND, NCAND)])
            return ()

        lax.fori_loop(0, RPW, row_body, (), unroll=False)

    return compact


def _make_decode():
    info = plsc.get_sparse_core_info()
    NW = info.num_cores * info.num_subcores
    RPW = BATCH // NW
    mesh = plsc.VectorSubcoreMesh(core_axis_name="c", subcore_axis_name="s")

    @functools.partial(
        pl.kernel, mesh=mesh,
        out_type=jax.ShapeDtypeStruct((BATCH * EMBED,), jnp.float32),
        scratch_types=[
            pltpu.VMEM((K,), jnp.int32),
            pltpu.VMEM((RPW * K + 16,), jnp.int32),
            pltpu.VMEM((RPW * K + 16,), jnp.float32),
            pltpu.VMEM((K, EMBED), jnp.float32),
            pltpu.VMEM((EMBED,), jnp.float32),
            pltpu.VMEM((EMBED,), jnp.float32),
            pltpu.SemaphoreType.DMA,
        ],
    )
    def decode(dec_hbm, feats_hbm, w_hbm, bias_hbm, out_hbm,
               fid_v, fall_v, wall_v, rows_v, bias_v, acc_v, sem):
        wid = lax.axis_index("s") * 2 + lax.axis_index("c")
        r0 = wid * RPW
        pltpu.sync_copy(bias_hbm, bias_v)
        pltpu.sync_copy(feats_hbm.at[pl.ds(r0 * K, RPW * K)],
                        fall_v.at[pl.ds(0, RPW * K)])
        pltpu.sync_copy(w_hbm.at[pl.ds(r0 * K, RPW * K)],
                        wall_v.at[pl.ds(0, RPW * K)])

        def row_body(r, _):
            row = r0 + r
            for t in range(K // 16):
                fid_v[pl.ds(t * 16, 16)] = fall_v[pl.ds(r * K + t * 16, 16)]
            pltpu.async_copy(dec_hbm.at[fid_v], rows_v, sem).wait()
            NB = 16
            for jb in range(EMBED // (16 * NB)):
                base = jb * 16 * NB
                accs = tuple(bias_v[pl.ds(base + t * 16, 16)]
                             for t in range(NB))

                def k_body(k, accs, base=base):
                    wk = (jnp.zeros((16,), jnp.float32)
                          + wall_v[pl.ds(r * K + k, 16)][0])
                    return tuple(
                        accs[t] + wk * rows_v[k, pl.ds(base + t * 16, 16)]
                        for t in range(NB))

                accs = lax.fori_loop(0, K, k_body, accs, unroll=False)
                for t in range(NB):
                    acc_v[pl.ds(base + t * 16, 16)] = accs[t]

            ss = jnp.zeros((16,), jnp.float32)
            for j in range(EMBED // 16):
                v = acc_v[pl.ds(j * 16, 16)]
                ss = ss + v * v
            tot_s = ss[0]
            for t in range(1, 16):
                tot_s = tot_s + ss[t]
            tot = jnp.zeros((16,), jnp.float32) + tot_s
            tot = jnp.maximum(tot, jnp.float32(1e-24))
            i = lax.bitcast_convert_type(tot, jnp.int32)
            y = lax.bitcast_convert_type(
                jnp.int32(0x5F3759DF) - (i >> 1), jnp.float32)
            half = jnp.float32(0.5) * tot
            for _ in range(4):
                y = y * (jnp.float32(1.5) - half * y * y)
            for j in range(EMBED // 16):
                acc_v[pl.ds(j * 16, 16)] = acc_v[pl.ds(j * 16, 16)] * y
            pltpu.sync_copy(acc_v, out_hbm.at[pl.ds(row * EMBED, EMBED)])
            return ()

        lax.fori_loop(0, RPW, row_body, (), unroll=False)

    return decode


def kernel(embed, enc_bias, enc_weight, dec_lookup):
    embed0 = (embed - enc_bias).astype(jnp.bfloat16)
    w_bf = enc_weight.astype(jnp.bfloat16)
    scores, tau, topchunks = _encode(embed0, w_bf)
    scoresC = scores.reshape(BATCH * NCHUNK, CW)
    cand_v, cand_f = _make_compact()(
        scoresC, topchunks.reshape(BATCH * K), tau.reshape(BATCH))
    weights, feats = _select(cand_v.reshape(BATCH, NCAND),
                             cand_f.reshape(BATCH, NCAND))
    embed1 = _make_decode()(dec_lookup, feats.reshape(BATCH * K),
                            weights.reshape(BATCH * K), enc_bias)
    return embed1.reshape(BATCH, EMBED)
